# baseline (speedup 1.0000x reference)
import functools

import jax
import jax.numpy as jnp
from jax import lax
from jax.experimental import pallas as pl
from jax.experimental.pallas import tpu as pltpu
from jax.experimental.pallas import tpu_sc as plsc

_NUM_ROWS = 1000000
_DIM = 32
_BATCH = 16384
_HIST = 50
_NW = 32
_BJ_W = 4
_NBLK = _HIST * _BJ_W

_mesh = plsc.VectorSubcoreMesh(core_axis_name="c", subcore_axis_name="s")


@functools.partial(
    pl.kernel,
    mesh=_mesh,
    out_type=jax.ShapeDtypeStruct((_HIST, 4, 128, 8, 128), jnp.float32),
    scratch_types=[
        pltpu.VMEM((_HIST, 512), jnp.int32),
        pltpu.VMEM((128, _DIM), jnp.float32),
        pltpu.VMEM((128, _DIM), jnp.float32),
        pltpu.VMEM((_DIM, 128), jnp.float32),
        pltpu.VMEM((_DIM, 128), jnp.float32),
        pltpu.SemaphoreType.DMA,
        pltpu.SemaphoreType.DMA,
        pltpu.SemaphoreType.DMA,
        pltpu.SemaphoreType.DMA,
    ],
    compiler_params=pltpu.CompilerParams(
        use_tc_tiling_on_sc=False, needs_layout_passes=False
    ),
)
def _gather_kernel(
    weight_hbm, xt_hbm, out_hbm,
    idx_v, rows0, rows1, tblk0, tblk1,
    gsem0, gsem1, wsem0, wsem1,
):
    wid = lax.axis_index("s") * 2 + lax.axis_index("c")
    col_base = wid * 512
    rows = (rows0, rows1)
    tblk = (tblk0, tblk1)
    gsem = (gsem0, gsem1)
    wsem = (wsem0, wsem1)
    iota16 = lax.iota(jnp.int32, 16)

    pltpu.sync_copy(xt_hbm.at[:, pl.ds(col_base, 512)], idx_v)

    def fire(k, b):
        h = k % _HIST
        bj = k // _HIST
        pltpu.async_copy(
            weight_hbm.at[idx_v.at[h, pl.ds(bj * 128, 128)]], rows[b], gsem[b]
        )

    def drain(b):
        pltpu.make_async_copy(
            weight_hbm.at[pl.ds(0, 128)], rows[b], gsem[b]
        ).wait()

    def transpose(b):
        src = rows[b]
        dst = tblk[b]

        def col_body(c, carry):
            cols = jnp.full((16,), c, jnp.int32)
            for l0 in range(0, 128, 16):
                v = plsc.load_gather(src, [iota16 + l0, cols])
                dst[c, pl.ds(l0, 16)] = v
            return carry

        lax.fori_loop(0, _DIM, col_body, 0)

    def start_wb(k, b):
        h = k % _HIST
        bj = k // _HIST
        for ci in range(4):
            pltpu.async_copy(
                tblk[b].at[pl.ds(ci * 8, 8)],
                out_hbm.at[h, ci, wid * _BJ_W + bj],
                wsem[b],
            )

    def wait_wb(b):
        for ci in range(4):
            pltpu.make_async_copy(
                tblk[b].at[pl.ds(ci * 8, 8)],
                out_hbm.at[0, ci, 0],
                wsem[b],
            ).wait()

    fire(0, 0)
    fire(1, 1)

    for k in (0, 1):
        b = k % 2
        drain(b)
        transpose(b)
        fire(k + 2, b)
        start_wb(k, b)

    def pair_body(p, carry):
        for b in range(2):
            k = 2 * p + 2 + b
            drain(b)
            wait_wb(b)
            transpose(b)
            fire(k + 2, b)
            start_wb(k, b)
        return carry

    lax.fori_loop(0, (_NBLK - 4) // 2, pair_body, 0)

    for k in (_NBLK - 2, _NBLK - 1):
        b = k % 2
        drain(b)
        wait_wb(b)
        transpose(b)
        start_wb(k, b)
    wait_wb(0)
    wait_wb(1)


def kernel(x, weight):
    xt = x.astype(jnp.int32).T
    o5 = _gather_kernel(weight, xt)
    return o5.transpose(2, 4, 0, 1, 3).reshape(_BATCH, _HIST, _DIM)
